# baseline (speedup 1.0000x reference)
import functools

import jax
import jax.numpy as jnp
from jax import lax
from jax.experimental import pallas as pl
from jax.experimental.pallas import tpu as pltpu
from jax.experimental.pallas import tpu_sc as plsc

_VOCAB = 100000
_DIM = 16
_BATCH = 16384
_HIST = 200

_NC = 2
_NS = 16
_NW = _NC * _NS
_B_PER_W = _BATCH // _NW
_CHUNK = 16
_N_CHUNKS = _B_PER_W // _CHUNK
_IDX_PER_CHUNK = _CHUNK * _HIST
_UNROLL = 8
_PROJ_BLK = 10000


def _proj_body(table_ref, w_ref, b_ref, out_ref):
    out_ref[...] = (
        jnp.sum(table_ref[...] * w_ref[...], axis=1, keepdims=True)
        + b_ref[...]
    )


def _project(table, W, b):
    return pl.pallas_call(
        _proj_body,
        grid=(_VOCAB // _PROJ_BLK,),
        in_specs=[
            pl.BlockSpec((_PROJ_BLK, _DIM), lambda i: (i, 0)),
            pl.BlockSpec((1, _DIM), lambda i: (0, 0)),
            pl.BlockSpec((1, 1), lambda i: (0, 0)),
        ],
        out_specs=pl.BlockSpec((_PROJ_BLK, 1), lambda i: (i, 0)),
        out_shape=jax.ShapeDtypeStruct((_VOCAB, 1), jnp.float32),
    )(table, W, b.reshape(1, 1))


@functools.partial(
    pl.kernel,
    out_type=jax.ShapeDtypeStruct((_BATCH,), jnp.float32),
    mesh=plsc.VectorSubcoreMesh(core_axis_name="c", subcore_axis_name="s"),
    compiler_params=pltpu.CompilerParams(
        needs_layout_passes=False, use_tc_tiling_on_sc=False
    ),
    scratch_types=[
        pltpu.VMEM((_VOCAB,), jnp.float32),
        pltpu.VMEM((2, _IDX_PER_CHUNK), jnp.int32),
        pltpu.VMEM((_B_PER_W,), jnp.float32),
        pltpu.SemaphoreType.DMA,
        pltpu.SemaphoreType.DMA,
        pltpu.SemaphoreType.DMA,
    ],
)
def _sc_pool(t_hbm, x_hbm, out_hbm, t_v, x_v, out_v, sem0, sem1, sem_t):
    wid = lax.axis_index("s") * _NC + lax.axis_index("c")
    base = wid * _B_PER_W * _HIST

    t_dma = pltpu.async_copy(t_hbm, t_v, sem_t)

    sems = (sem0, sem1)

    def start_fetch(c):
        return pltpu.async_copy(
            x_hbm.at[pl.ds(base + c * _IDX_PER_CHUNK, _IDX_PER_CHUNK)],
            x_v.at[c % 2],
            sems[c % 2],
        )

    lane_offs = lax.iota(jnp.int32, 16) * _HIST
    zero = jnp.zeros((16,), jnp.float32)

    dmas = [start_fetch(0), None]
    t_dma.wait()

    for c in range(_N_CHUNKS):
        if c + 1 < _N_CHUNKS:
            dmas[(c + 1) % 2] = start_fetch(c + 1)
        dmas[c % 2].wait()
        xc = x_v.at[c % 2]

        def inner(i, carry, xc=xc):
            a0, a1, a2, a3, offs = carry
            for u in range(_UNROLL):
                idx = plsc.load_gather(xc, [offs + u])
                val = plsc.load_gather(t_v, [idx])
                if u % 4 == 0:
                    a0 = a0 + val
                elif u % 4 == 1:
                    a1 = a1 + val
                elif u % 4 == 2:
                    a2 = a2 + val
                else:
                    a3 = a3 + val
            return a0, a1, a2, a3, offs + _UNROLL

        a0, a1, a2, a3, _ = lax.fori_loop(
            0, _HIST // _UNROLL, inner, (zero, zero, zero, zero, lane_offs)
        )
        z = ((a0 + a1) + (a2 + a3)) * (1.0 / _HIST)
        out_v[pl.ds(c * _CHUNK, _CHUNK)] = 1.0 / (1.0 + jnp.exp(-z))

    pltpu.sync_copy(out_v, out_hbm.at[pl.ds(wid * _B_PER_W, _B_PER_W)])


def kernel(x, table, W, b):
    t = _project(table, W, b).reshape(_VOCAB)
    out = _sc_pool(t, x.reshape(_BATCH * _HIST))
    return out.reshape(_BATCH, 1)


# speedup vs baseline: 1.1880x; 1.1880x over previous
import functools

import jax
import jax.numpy as jnp
from jax import lax
from jax.experimental import pallas as pl
from jax.experimental.pallas import tpu as pltpu
from jax.experimental.pallas import tpu_sc as plsc

_VOCAB = 100000
_DIM = 16
_BATCH = 16384
_HIST = 200

_NC = 2
_NS = 16
_NW = _NC * _NS
_B_PER_W = _BATCH // _NW
_CHUNK = 16
_N_CHUNKS = _B_PER_W // _CHUNK
_UNROLL = 8
_T_COLS = _VOCAB // 8


def _proj_body(table_ref, w_ref, b_ref, out_ref):
    w128 = jnp.tile(w_ref[...], (1, 8))
    prod = table_ref[...] * w128
    c = lax.broadcasted_iota(jnp.int32, (128, 8), 0)
    j = lax.broadcasted_iota(jnp.int32, (128, 8), 1)
    sel = jnp.where(c // 16 == j, 1.0, 0.0)
    out_ref[...] = (
        lax.dot_general(
            sel, prod,
            dimension_numbers=(((0,), (1,)), ((), ())),
            preferred_element_type=jnp.float32,
        )
        + b_ref[...]
    )


def _project(table, W, b):
    return pl.pallas_call(
        _proj_body,
        out_shape=jax.ShapeDtypeStruct((8, _T_COLS), jnp.float32),
    )(table.reshape(_T_COLS, 128), W, b.reshape(1, 1))


@functools.partial(
    pl.kernel,
    out_type=jax.ShapeDtypeStruct((_BATCH,), jnp.float32),
    mesh=plsc.VectorSubcoreMesh(core_axis_name="c", subcore_axis_name="s"),
    compiler_params=pltpu.CompilerParams(
        needs_layout_passes=False, use_tc_tiling_on_sc=False
    ),
    scratch_types=[
        pltpu.VMEM((8, _T_COLS), jnp.float32),
        pltpu.VMEM((2, _CHUNK, _HIST), jnp.int32),
        pltpu.VMEM((_B_PER_W,), jnp.float32),
        pltpu.SemaphoreType.DMA,
        pltpu.SemaphoreType.DMA,
        pltpu.SemaphoreType.DMA,
    ],
)
def _sc_pool(t_hbm, x_hbm, out_hbm, t_v, x_v, out_v, sem0, sem1, sem_t):
    wid = lax.axis_index("s") * _NC + lax.axis_index("c")
    row0 = wid * _B_PER_W

    t_dma = pltpu.async_copy(t_hbm, t_v, sem_t)

    sems = (sem0, sem1)

    def start_fetch(c):
        return pltpu.async_copy(
            x_hbm.at[pl.ds(row0 + c * _CHUNK, _CHUNK), :],
            x_v.at[c % 2],
            sems[c % 2],
        )

    lane = lax.iota(jnp.int32, 16)
    zero = jnp.zeros((16,), jnp.float32)
    izero = jnp.zeros((16,), jnp.int32)

    dmas = [start_fetch(0), None]
    t_dma.wait()

    for c in range(_N_CHUNKS):
        if c + 1 < _N_CHUNKS:
            dmas[(c + 1) % 2] = start_fetch(c + 1)
        dmas[c % 2].wait()
        xc = x_v.at[c % 2]

        def inner(i, carry, xc=xc):
            a0, a1, a2, a3, l = carry
            for u in range(_UNROLL):
                idx = plsc.load_gather(xc, [lane, l + u])
                val = plsc.load_gather(
                    t_v, [lax.bitwise_and(idx, 7), lax.shift_right_logical(idx, 3)]
                )
                if u % 4 == 0:
                    a0 = a0 + val
                elif u % 4 == 1:
                    a1 = a1 + val
                elif u % 4 == 2:
                    a2 = a2 + val
                else:
                    a3 = a3 + val
            return a0, a1, a2, a3, l + _UNROLL

        a0, a1, a2, a3, _ = lax.fori_loop(
            0, _HIST // _UNROLL, inner, (zero, zero, zero, zero, izero)
        )
        z = ((a0 + a1) + (a2 + a3)) * (1.0 / _HIST)
        out_v[pl.ds(c * _CHUNK, _CHUNK)] = 1.0 / (1.0 + jnp.exp(-z))

    pltpu.sync_copy(out_v, out_hbm.at[pl.ds(row0, _B_PER_W)])


def kernel(x, table, W, b):
    t = _project(table, W, b)
    out = _sc_pool(t, x)
    return out.reshape(_BATCH, 1)


# speedup vs baseline: 1.1927x; 1.0040x over previous
import functools

import jax
import jax.numpy as jnp
from jax import lax
from jax.experimental import pallas as pl
from jax.experimental.pallas import tpu as pltpu
from jax.experimental.pallas import tpu_sc as plsc

_VOCAB = 100000
_DIM = 16
_BATCH = 16384
_HIST = 200

_NC = 2
_NS = 16
_NW = _NC * _NS
_B_PER_W = _BATCH // _NW
_CHUNK = 16
_N_CHUNKS = _B_PER_W // _CHUNK
_UNROLL = 8
_T_COLS = _VOCAB // 8
_T_PITCH = 12544


def _proj_body(table_ref, w_ref, b_ref, out_ref):
    w128 = jnp.tile(w_ref[...], (1, 8))
    prod = table_ref[...] * w128
    c = lax.broadcasted_iota(jnp.int32, (128, 8), 0)
    j = lax.broadcasted_iota(jnp.int32, (128, 8), 1)
    sel = jnp.where(c // 16 == j, 1.0, 0.0)
    out_ref[...] = (
        lax.dot_general(
            sel, prod,
            dimension_numbers=(((0,), (1,)), ((), ())),
            preferred_element_type=jnp.float32,
        )
        + b_ref[...]
    )


def _project(table, W, b):
    return pl.pallas_call(
        _proj_body,
        out_shape=jax.ShapeDtypeStruct((8, _T_COLS), jnp.float32),
    )(table.reshape(_T_COLS, 128), W, b.reshape(1, 1))


@functools.partial(
    pl.kernel,
    out_type=jax.ShapeDtypeStruct((_BATCH,), jnp.float32),
    mesh=plsc.VectorSubcoreMesh(core_axis_name="c", subcore_axis_name="s"),
    compiler_params=pltpu.CompilerParams(
        needs_layout_passes=False, use_tc_tiling_on_sc=False
    ),
    scratch_types=[
        pltpu.VMEM((8, _T_PITCH), jnp.float32),
        pltpu.VMEM((2, _CHUNK, _HIST), jnp.int32),
        pltpu.VMEM((_B_PER_W,), jnp.float32),
        pltpu.SemaphoreType.DMA,
        pltpu.SemaphoreType.DMA,
        pltpu.SemaphoreType.DMA,
    ],
)
def _sc_pool(t_hbm, x_hbm, out_hbm, t_v, x_v, out_v, sem0, sem1, sem_t):
    wid = lax.axis_index("s") * _NC + lax.axis_index("c")
    row0 = wid * _B_PER_W

    t_dma = pltpu.async_copy(t_hbm, t_v, sem_t)

    sems = (sem0, sem1)

    def start_fetch(c):
        return pltpu.async_copy(
            x_hbm.at[pl.ds(row0 + c * _CHUNK, _CHUNK), :],
            x_v.at[c % 2],
            sems[c % 2],
        )

    lane = lax.iota(jnp.int32, 16)
    zero = jnp.zeros((16,), jnp.float32)
    izero = jnp.zeros((16,), jnp.int32)

    dmas = [start_fetch(0), None]
    t_dma.wait()

    for c in range(_N_CHUNKS):
        if c + 1 < _N_CHUNKS:
            dmas[(c + 1) % 2] = start_fetch(c + 1)
        dmas[c % 2].wait()
        xc = x_v.at[c % 2]

        def inner(i, carry, xc=xc):
            a0, a1, a2, a3, l = carry
            for u in range(_UNROLL):
                idx = plsc.load_gather(xc, [lane, l + u])
                val = plsc.load_gather(
                    t_v, [lax.bitwise_and(idx, 7), lax.shift_right_logical(idx, 3)]
                )
                if u % 4 == 0:
                    a0 = a0 + val
                elif u % 4 == 1:
                    a1 = a1 + val
                elif u % 4 == 2:
                    a2 = a2 + val
                else:
                    a3 = a3 + val
            return a0, a1, a2, a3, l + _UNROLL

        a0, a1, a2, a3, _ = lax.fori_loop(
            0, _HIST // _UNROLL, inner, (zero, zero, zero, zero, izero)
        )
        z = ((a0 + a1) + (a2 + a3)) * (1.0 / _HIST)
        out_v[pl.ds(c * _CHUNK, _CHUNK)] = 1.0 / (1.0 + jnp.exp(-z))

    pltpu.sync_copy(out_v, out_hbm.at[pl.ds(row0, _B_PER_W)])


def kernel(x, table, W, b):
    t = _project(table, W, b)
    t = jnp.pad(t, ((0, 0), (0, _T_PITCH - _T_COLS)))
    out = _sc_pool(t, x)
    return out.reshape(_BATCH, 1)
